# baseline (speedup 1.0000x reference)
import functools

import jax
import jax.numpy as jnp
from jax import lax
from jax.experimental import pallas as pl
from jax.experimental.pallas import tpu as pltpu
from jax.experimental.pallas import tpu_sc as plsc

N = 1024
D = 256
K = 8192
KC = 2048


def _argmin_body(a_sq_ref, z_ref, cb_ref, idx_ref, dist_ref):
    k = pl.program_id(0)
    cb = cb_ref[...]
    zm2 = z_ref[...] * (-2.0)
    ab2 = lax.dot_general(
        cb, zm2, (((1,), (1,)), ((), ())),
        preferred_element_type=jnp.float32)
    bsq = jnp.sum(cb * cb, axis=1, keepdims=True)
    d = (a_sq_ref[...] + ab2) + bsq
    m = jnp.min(d, axis=0, keepdims=True)
    row_f = lax.broadcasted_iota(jnp.int32, (KC, 1), 0).astype(jnp.float32)
    colf = jnp.where(d == m, row_f, jnp.float32(K))
    lif = jnp.min(colf, axis=0, keepdims=True)
    li = lif.astype(jnp.int32) + k * KC

    @pl.when(k == 0)
    def _():
        dist_ref[...] = m
        idx_ref[...] = li

    @pl.when(k > 0)
    def _():
        p = m < dist_ref[...]
        dist_ref[...] = jnp.where(p, m, dist_ref[...])
        idx_ref[...] = jnp.where(p, li, idx_ref[...])


def _argmin_call(a_sq, z, codebook):
    return pl.pallas_call(
        _argmin_body,
        grid=(K // KC,),
        in_specs=[
            pl.BlockSpec((1, N), lambda k: (0, 0)),
            pl.BlockSpec((N, D), lambda k: (0, 0)),
            pl.BlockSpec((KC, D), lambda k: (k, 0)),
        ],
        out_specs=[
            pl.BlockSpec((1, N), lambda k: (0, 0)),
            pl.BlockSpec((1, N), lambda k: (0, 0)),
        ],
        out_shape=[
            jax.ShapeDtypeStruct((1, N), jnp.int32),
            jax.ShapeDtypeStruct((1, N), jnp.float32),
        ],
        compiler_params=pltpu.CompilerParams(
            dimension_semantics=("arbitrary",)),
    )(a_sq, z, codebook)


@functools.cache
def _make_sc_gather(num_rows):
    mesh = plsc.VectorSubcoreMesh(core_axis_name="c", subcore_axis_name="s",
                                  num_cores=2, num_subcores=16)
    nc, ns = mesh.num_cores, mesh.num_subcores
    nw = nc * ns
    b_per_w = num_rows // nw

    @functools.partial(
        pl.kernel,
        mesh=mesh,
        out_type=jax.ShapeDtypeStruct((num_rows, D), jnp.float32),
        scratch_types=[
            pltpu.VMEM((b_per_w,), jnp.int32),
            pltpu.VMEM((b_per_w, D), jnp.float32),
            pltpu.SemaphoreType.DMA,
        ],
    )
    def gather(table_hbm, idx_hbm, out_hbm, idx_v, rows_v, sem):
        wid = lax.axis_index("s") * nc + lax.axis_index("c")
        base = wid * b_per_w
        pltpu.sync_copy(idx_hbm.at[pl.ds(base, b_per_w)], idx_v)
        pltpu.async_copy(table_hbm.at[idx_v], rows_v, sem).wait()
        pltpu.sync_copy(rows_v, out_hbm.at[pl.ds(base, b_per_w)])

    return gather


def kernel(z_e, codebook):
    hh, ww, d = z_e.shape
    z = z_e.reshape(hh * ww, d)
    a_sq = jnp.sum(z * z, axis=1)[None, :]
    idx2, dist2 = _argmin_call(a_sq, z, codebook)
    indices_flat = idx2.reshape(hh * ww)
    min_distances = dist2.reshape(hh * ww)
    z_q_flat = _make_sc_gather(hh * ww)(codebook, indices_flat)
    return (z_q_flat.reshape(hh, ww, d),
            indices_flat.reshape(hh, ww),
            min_distances)


# speedup vs baseline: 1.1776x; 1.1776x over previous
import functools

import jax
import jax.numpy as jnp
from jax import lax
from jax.experimental import pallas as pl
from jax.experimental.pallas import tpu as pltpu
from jax.experimental.pallas import tpu_sc as plsc

N = 1024
D = 256
K = 8192
KC = 2048


def _argmin_body(a_sq_ref, z_ref, cb_ref, idx_ref, dist_ref, zq_ref):
    k = pl.program_id(0)
    cb = cb_ref[...]
    zm2 = z_ref[...] * (-2.0)
    ab2 = lax.dot_general(
        cb, zm2, (((1,), (1,)), ((), ())),
        preferred_element_type=jnp.float32)
    bsq = jnp.sum(cb * cb, axis=1, keepdims=True)
    d = (a_sq_ref[...] + ab2) + bsq
    m = jnp.min(d, axis=0, keepdims=True)
    row_f = lax.broadcasted_iota(jnp.int32, (KC, 1), 0).astype(jnp.float32)
    colf = jnp.where(d == m, row_f, jnp.float32(K))
    lif = jnp.min(colf, axis=0, keepdims=True)
    li = lif.astype(jnp.int32) + k * KC

    oh = jnp.where(row_f == lif, 1.0, 0.0)
    delta = lax.dot_general(
        oh, cb, (((0,), (0,)), ((), ())),
        preferred_element_type=jnp.float32)

    @pl.when(k == 0)
    def _():
        dist_ref[...] = m
        idx_ref[...] = li
        zq_ref[...] = delta

    @pl.when(k > 0)
    def _():
        p = m < dist_ref[...]
        dist_ref[...] = jnp.where(p, m, dist_ref[...])
        idx_ref[...] = jnp.where(p, li, idx_ref[...])
        pcol = jnp.transpose(p)
        zq_ref[...] = jnp.where(pcol, delta, zq_ref[...])


def _argmin_call(a_sq, z, codebook):
    return pl.pallas_call(
        _argmin_body,
        grid=(K // KC,),
        in_specs=[
            pl.BlockSpec((1, N), lambda k: (0, 0)),
            pl.BlockSpec((N, D), lambda k: (0, 0)),
            pl.BlockSpec((KC, D), lambda k: (k, 0)),
        ],
        out_specs=[
            pl.BlockSpec((1, N), lambda k: (0, 0)),
            pl.BlockSpec((1, N), lambda k: (0, 0)),
            pl.BlockSpec((N, D), lambda k: (0, 0)),
        ],
        out_shape=[
            jax.ShapeDtypeStruct((1, N), jnp.int32),
            jax.ShapeDtypeStruct((1, N), jnp.float32),
            jax.ShapeDtypeStruct((N, D), jnp.float32),
        ],
        compiler_params=pltpu.CompilerParams(
            dimension_semantics=("arbitrary",)),
    )(a_sq, z, codebook)


@functools.cache
def _make_sc_gather(num_rows):
    mesh = plsc.VectorSubcoreMesh(core_axis_name="c", subcore_axis_name="s",
                                  num_cores=2, num_subcores=16)
    nc, ns = mesh.num_cores, mesh.num_subcores
    nw = nc * ns
    b_per_w = num_rows // nw

    @functools.partial(
        pl.kernel,
        mesh=mesh,
        out_type=jax.ShapeDtypeStruct((num_rows, D), jnp.float32),
        scratch_types=[
            pltpu.VMEM((b_per_w,), jnp.int32),
            pltpu.VMEM((b_per_w, D), jnp.float32),
            pltpu.SemaphoreType.DMA,
        ],
    )
    def gather(table_hbm, idx_hbm, out_hbm, idx_v, rows_v, sem):
        wid = lax.axis_index("s") * nc + lax.axis_index("c")
        base = wid * b_per_w
        pltpu.sync_copy(idx_hbm.at[pl.ds(base, b_per_w)], idx_v)
        pltpu.async_copy(table_hbm.at[idx_v], rows_v, sem).wait()
        pltpu.sync_copy(rows_v, out_hbm.at[pl.ds(base, b_per_w)])

    return gather


def kernel(z_e, codebook):
    hh, ww, d = z_e.shape
    z = z_e.reshape(hh * ww, d)
    a_sq = jnp.sum(z * z, axis=1)[None, :]
    idx2, dist2, z_q_flat = _argmin_call(a_sq, z, codebook)
    indices_flat = idx2.reshape(hh * ww)
    min_distances = dist2.reshape(hh * ww)
    return (z_q_flat.reshape(hh, ww, d),
            indices_flat.reshape(hh, ww),
            min_distances)
